# baseline (speedup 1.0000x reference)
import jax
import jax.numpy as jnp
from jax.experimental import pallas as pl

_STRIDE = 32
_NKEEP = 64


def _gather_body(x_ref, o_ref):
    x = x_ref[...]
    r = x.shape[0]
    o_ref[...] = x.reshape(r, _NKEEP, _STRIDE)[:, :, 0]


def kernel(inputs):
    b, s, f = inputs.shape
    flat = inputs.reshape(b * s, f)
    rows = 512
    out = pl.pallas_call(
        _gather_body,
        grid=(b * s // rows,),
        in_specs=[pl.BlockSpec((rows, f), lambda i: (i, 0))],
        out_specs=pl.BlockSpec((rows, _NKEEP), lambda i: (i, 0)),
        out_shape=jax.ShapeDtypeStruct((b * s, _NKEEP), inputs.dtype),
    )(flat)
    return out.reshape(b, s, _NKEEP)
